# Initial kernel scaffold; baseline (speedup 1.0000x reference)
import jax
import jax.numpy as jnp
from jax.experimental import pallas as pl


def kernel(x, edge_index, edge_attr, W1, b1, Wmu, bmu, Wls, bls):
    raise NotImplementedError("write your pallas kernel here")


# speedup vs baseline: 7.8589x; 7.8589x over previous
import functools

import jax
import jax.numpy as jnp
from jax import lax
from jax.experimental import pallas as pl
from jax.experimental.pallas import tpu as pltpu
from jax.experimental.pallas import tpu_sc as plsc

N = 10000
E = 320000
EP = 327680
NACC = 10240
DIN = 128
HID = 256
OUT = 128
NC, NS = 2, 16
CHUNK = 256
G = CHUNK // 128


def _deg_body(dsts, ews, out, dst_v, ew_v, zbuf1, acc1):
    c = lax.axis_index("c")
    s = lax.axis_index("s")
    wid = c * NS + s
    tp = EP // (NC * NS)
    tpr = tp // 128
    nps = NACC // NS

    @pl.loop(0, nps // 16)
    def _(r):
        zbuf1[pl.ds(r * 16, 16)] = jnp.zeros((16,), jnp.float32)

    pltpu.sync_copy(zbuf1, acc1.at[pl.ds(s * nps, nps)])
    plsc.subcore_barrier()

    @pl.loop(0, tp // CHUNK)
    def _(g):
        pltpu.sync_copy(dsts.at[pl.ds(wid * tpr + g * G, G)], dst_v)
        pltpu.sync_copy(ews.at[pl.ds(wid * tp + g * CHUNK, CHUNK)], ew_v)
        for j in range(G):
            pltpu.sync_copy(ew_v.at[pl.ds(j * 128, 128)],
                            acc1.at[dst_v.at[j]], add=True)

    plsc.subcore_barrier()
    pltpu.sync_copy(acc1.at[pl.ds(s * nps, nps)],
                    out.at[c, pl.ds(s * nps, nps)])


def _deg_call(dst2d, ewp):
    mesh = plsc.VectorSubcoreMesh(core_axis_name="c", subcore_axis_name="s", num_cores=NC, num_subcores=NS)
    return pl.kernel(
        _deg_body,
        out_type=jax.ShapeDtypeStruct((NC, NACC), jnp.float32),
        mesh=mesh,
        scratch_types=[
            pltpu.VMEM((G, 128), jnp.int32),
            pltpu.VMEM((CHUNK,), jnp.float32),
            pltpu.VMEM((NACC // NS,), jnp.float32),
            pltpu.VMEM_SHARED((NACC,), jnp.float32),
        ],
        compiler_params=pltpu.CompilerParams(needs_layout_passes=False),
        name="sc_deg",
    )(dst2d, ewp)


def _agg_body(split_edges, tab, srcs, dsts, ews, dis_hbm, y_out,
              dis_v, src_v, dst_v, idx_v, ew_v, nrm_v, rows, acc, sem):
    c = lax.axis_index("c")
    s = lax.axis_index("s")

    @pl.loop(0, CHUNK)
    def _(r):
        for k in range(8):
            rows[r, pl.ds(k * 16, 16)] = jnp.zeros((16,), jnp.float32)

    pltpu.sync_copy(rows, acc.at[pl.ds(s * (NACC // NS), CHUNK)])
    pltpu.sync_copy(rows, acc.at[pl.ds(s * (NACC // NS) + CHUNK, CHUNK)])
    pltpu.sync_copy(rows.at[pl.ds(0, 128)],
                    acc.at[pl.ds(s * (NACC // NS) + 2 * CHUNK, 128)])

    pltpu.sync_copy(dis_hbm, dis_v)
    plsc.subcore_barrier()

    if split_edges:
        tp = EP // (NC * NS)
        tpr = tp // 128
        srow = (c * NS + s) * tpr
        drow = (c * NS + s) * tpr
        ebase = (c * NS + s) * tp
    else:
        tp = EP // NS
        tpr = tp // 128
        srow = s * tpr
        drow = s * tpr
        ebase = s * tp
    off = 0 if split_edges else c * N

    @pl.loop(0, tp // CHUNK)
    def _(g):
        pltpu.sync_copy(srcs.at[pl.ds(srow + g * G, G)], src_v)
        pltpu.sync_copy(dsts.at[pl.ds(drow + g * G, G)], dst_v)
        pltpu.sync_copy(ews.at[pl.ds(ebase + g * CHUNK, CHUNK)], ew_v)

        for i in range(CHUNK // 16):
            s16 = src_v[i // 8, pl.ds((i % 8) * 16, 16)]
            d16 = dst_v[i // 8, pl.ds((i % 8) * 16, 16)]
            e16 = ew_v[pl.ds(i * 16, 16)]
            idx_v[i // 8, pl.ds((i % 8) * 16, 16)] = s16 + off
            nrm_v[pl.ds(i * 16, 16)] = (
                plsc.load_gather(dis_v, [s16]) * e16
                * plsc.load_gather(dis_v, [d16])
            )

        cps = [
            pltpu.async_copy(tab.at[idx_v.at[j]],
                             rows.at[pl.ds(j * 128, 128)], sem)
            for j in range(G)
        ]
        for cp in cps:
            cp.wait()

        @plsc.parallel_loop(0, CHUNK, unroll=8)
        def _(e):
            sp = plsc.load_gather(nrm_v, [jnp.full((16,), e, jnp.int32)])
            for k in range(8):
                rows[e, pl.ds(k * 16, 16)] = rows[e, pl.ds(k * 16, 16)] * sp

        for j in range(G):
            pltpu.sync_copy(rows.at[pl.ds(j * 128, 128)],
                            acc.at[dst_v.at[j]], add=True)

    plsc.subcore_barrier()
    pltpu.sync_copy(acc.at[pl.ds(s * (NACC // NS), NACC // NS)],
                    y_out.at[c, pl.ds(s * (NACC // NS), NACC // NS)])


def _agg_call(split_edges, tab, srcs, dsts, ews, dis1):
    mesh = plsc.VectorSubcoreMesh(core_axis_name="c", subcore_axis_name="s", num_cores=NC, num_subcores=NS)
    return pl.kernel(
        functools.partial(_agg_body, split_edges),
        out_type=jax.ShapeDtypeStruct((NC, NACC, 128), jnp.float32),
        mesh=mesh,
        scratch_types=[
            pltpu.VMEM((NACC,), jnp.float32),
            pltpu.VMEM((G, 128), jnp.int32),
            pltpu.VMEM((G, 128), jnp.int32),
            pltpu.VMEM((G, 128), jnp.int32),
            pltpu.VMEM((CHUNK,), jnp.float32),
            pltpu.VMEM((CHUNK,), jnp.float32),
            pltpu.VMEM((CHUNK, 128), jnp.float32),
            pltpu.VMEM_SHARED((NACC, 128), jnp.float32),
            pltpu.SemaphoreType.DMA,
        ],
        compiler_params=pltpu.CompilerParams(needs_layout_passes=False),
        name="sc_agg",
    )(tab, srcs, dsts, ews, dis1)


def _dis_body(deg_ref, dis_ref):
    d = jnp.sum(deg_ref[...], axis=0, keepdims=True) + 1.0
    dis_ref[...] = lax.rsqrt(d)


def _dis_call(degs):
    return pl.pallas_call(
        _dis_body,
        grid=(NACC // 512,),
        in_specs=[pl.BlockSpec((NC, 512), lambda i: (0, i))],
        out_specs=pl.BlockSpec((1, 512), lambda i: (0, i)),
        out_shape=jax.ShapeDtypeStruct((1, NACC), jnp.float32),
    )(degs)


_RB = 400


def _mid_body(y1_ref, x_ref, dis_ref, w1_ref, b1_ref, hs_ref):
    d = dis_ref[...]
    agg = y1_ref[0] + y1_ref[1] + (d * d) * x_ref[...]
    h = jnp.dot(agg, w1_ref[...], preferred_element_type=jnp.float32) + b1_ref[...]
    hs_ref[0] = jnp.maximum(h, 0.0)


def _mid_call(y1, x, dis2d, W1, b1r):
    return pl.pallas_call(
        _mid_body,
        grid=(N // _RB, 2),
        in_specs=[
            pl.BlockSpec((NC, _RB, 128), lambda i, c: (0, i, 0)),
            pl.BlockSpec((_RB, DIN), lambda i, c: (i, 0)),
            pl.BlockSpec((_RB, 1), lambda i, c: (i, 0)),
            pl.BlockSpec((DIN, 128), lambda i, c: (0, c)),
            pl.BlockSpec((1, 128), lambda i, c: (0, c)),
        ],
        out_specs=pl.BlockSpec((1, _RB, 128), lambda i, c: (c, i, 0)),
        out_shape=jax.ShapeDtypeStruct((2, N, 128), jnp.float32),
    )(y1, x, dis2d, W1, b1r)


def _fin_body(y2_ref, hs_ref, dis_ref, wmu_ref, bmu_ref, wls_ref, bls_ref,
              mu_ref, ls_ref):
    d = dis_ref[...]
    d2 = d * d
    aa = y2_ref[0] + d2 * hs_ref[0]
    ab = y2_ref[1] + d2 * hs_ref[1]
    mu_ref[...] = (
        jnp.dot(aa, wmu_ref[0:128], preferred_element_type=jnp.float32)
        + jnp.dot(ab, wmu_ref[128:256], preferred_element_type=jnp.float32)
        + bmu_ref[...]
    )
    ls_ref[...] = (
        jnp.dot(aa, wls_ref[0:128], preferred_element_type=jnp.float32)
        + jnp.dot(ab, wls_ref[128:256], preferred_element_type=jnp.float32)
        + bls_ref[...]
    )


def _fin_call(y2, hs, dis2d, Wmu, bmur, Wls, blsr):
    return pl.pallas_call(
        _fin_body,
        grid=(N // _RB,),
        in_specs=[
            pl.BlockSpec((NC, _RB, 128), lambda i: (0, i, 0)),
            pl.BlockSpec((NC, _RB, 128), lambda i: (0, i, 0)),
            pl.BlockSpec((_RB, 1), lambda i: (i, 0)),
            pl.BlockSpec((HID, OUT), lambda i: (0, 0)),
            pl.BlockSpec((1, OUT), lambda i: (0, 0)),
            pl.BlockSpec((HID, OUT), lambda i: (0, 0)),
            pl.BlockSpec((1, OUT), lambda i: (0, 0)),
        ],
        out_specs=[
            pl.BlockSpec((_RB, OUT), lambda i: (i, 0)),
            pl.BlockSpec((_RB, OUT), lambda i: (i, 0)),
        ],
        out_shape=[
            jax.ShapeDtypeStruct((N, OUT), jnp.float32),
            jax.ShapeDtypeStruct((N, OUT), jnp.float32),
        ],
    )(y2, hs, dis2d, Wmu, bmur, Wls, blsr)


@jax.jit
def kernel(x, edge_index, edge_attr, W1, b1, Wmu, bmu, Wls, bls):
    src = edge_index[0].astype(jnp.int32)
    dst = edge_index[1].astype(jnp.int32)
    pad = EP - E
    srcp = jnp.concatenate([src, jnp.zeros((pad,), jnp.int32)])
    dstp = jnp.concatenate([dst, jnp.full((pad,), N, jnp.int32)])
    ewp = jnp.concatenate([edge_attr, jnp.zeros((pad,), jnp.float32)])

    src1 = srcp.reshape(EP // 128, 128)
    dst2d = dstp.reshape(EP // 128, 128)

    degs = _deg_call(dst2d, ewp)
    dis = _dis_call(degs)
    dis1 = dis.reshape(NACC)
    dis2d = dis.reshape(NACC, 1)

    y1p = _agg_call(True, x, src1, dst2d, ewp, dis1)
    hs = _mid_call(y1p, x, dis2d[:N], W1, b1.reshape(1, HID))
    tab2 = hs.reshape(2 * N, 128)
    y2 = _agg_call(False, tab2, src1, dst2d, ewp, dis1)
    mu, ls = _fin_call(y2, hs, dis2d[:N], Wmu, bmu.reshape(1, OUT),
                       Wls, bls.reshape(1, OUT))
    return (mu, ls)


# speedup vs baseline: 8.4527x; 1.0756x over previous
import functools

import jax
import jax.numpy as jnp
from jax import lax
from jax.experimental import pallas as pl
from jax.experimental.pallas import tpu as pltpu
from jax.experimental.pallas import tpu_sc as plsc

N = 10000
E = 320000
EP = 327680
NACC = 10240
DIN = 128
HID = 256
OUT = 128
NC, NS = 2, 16
CHUNK = 256
G = CHUNK // 128
AC = 128


def _deg_body(dsts, ews, out, dst_v, ew_v, zbuf1, acc1):
    c = lax.axis_index("c")
    s = lax.axis_index("s")
    wid = c * NS + s
    tp = EP // (NC * NS)
    tpr = tp // 128
    nps = NACC // NS

    @pl.loop(0, nps // 16)
    def _(r):
        zbuf1[pl.ds(r * 16, 16)] = jnp.zeros((16,), jnp.float32)

    pltpu.sync_copy(zbuf1, acc1.at[pl.ds(s * nps, nps)])
    plsc.subcore_barrier()

    @pl.loop(0, tp // CHUNK)
    def _(g):
        pltpu.sync_copy(dsts.at[pl.ds(wid * tpr + g * G, G)], dst_v)
        pltpu.sync_copy(ews.at[pl.ds(wid * tp + g * CHUNK, CHUNK)], ew_v)
        for j in range(G):
            pltpu.sync_copy(ew_v.at[pl.ds(j * 128, 128)],
                            acc1.at[dst_v.at[j]], add=True)

    plsc.subcore_barrier()
    pltpu.sync_copy(acc1.at[pl.ds(s * nps, nps)],
                    out.at[c, pl.ds(s * nps, nps)])


def _deg_call(dst2d, ewp):
    mesh = plsc.VectorSubcoreMesh(core_axis_name="c", subcore_axis_name="s", num_cores=NC, num_subcores=NS)
    return pl.kernel(
        _deg_body,
        out_type=jax.ShapeDtypeStruct((NC, NACC), jnp.float32),
        mesh=mesh,
        scratch_types=[
            pltpu.VMEM((G, 128), jnp.int32),
            pltpu.VMEM((CHUNK,), jnp.float32),
            pltpu.VMEM((NACC // NS,), jnp.float32),
            pltpu.VMEM_SHARED((NACC,), jnp.float32),
        ],
        compiler_params=pltpu.CompilerParams(needs_layout_passes=False),
        name="sc_deg",
    )(dst2d, ewp)


def _agg_body(split_edges, tab, srcs, dsts, ews, dis_hbm, y_out,
              dis_v, src_v, dst_v, idx_v, ew_v, nrm_v, rows, acc, gsem, ssem):
    c = lax.axis_index("c")
    s = lax.axis_index("s")

    @pl.loop(0, AC)
    def _(r):
        for b in range(2):
            for k in range(8):
                rows[b, r, pl.ds(k * 16, 16)] = jnp.zeros((16,), jnp.float32)

    for j in range(NACC // NS // AC):
        pltpu.sync_copy(rows.at[j % 2],
                        acc.at[pl.ds(s * (NACC // NS) + j * AC, AC)])

    pltpu.sync_copy(dis_hbm, dis_v)
    plsc.subcore_barrier()

    if split_edges:
        tp = EP // (NC * NS)
        base = (c * NS + s) * tp
        srow = base // 128
        drow = srow
    else:
        tp = EP // NS
        base = s * tp
        srow = base // 128
        drow = srow
    off = 0 if split_edges else c * N

    def stage_a(g, b):
        pltpu.sync_copy(srcs.at[pl.ds(srow + g, 1)], src_v.at[pl.ds(b, 1)])
        pltpu.sync_copy(dsts.at[pl.ds(drow + g, 1)], dst_v.at[pl.ds(b, 1)])
        pltpu.sync_copy(ews.at[pl.ds(base + g * AC, AC)], ew_v.at[b])
        for i in range(AC // 16):
            s16 = src_v[b, pl.ds(i * 16, 16)]
            d16 = dst_v[b, pl.ds(i * 16, 16)]
            e16 = ew_v[b, pl.ds(i * 16, 16)]
            if not split_edges:
                idx_v[b, pl.ds(i * 16, 16)] = s16 + off
            nrm_v[b, pl.ds(i * 16, 16)] = (
                plsc.load_gather(dis_v, [s16]) * e16
                * plsc.load_gather(dis_v, [d16])
            )
        gidx = src_v if split_edges else idx_v
        return pltpu.async_copy(tab.at[gidx.at[b]], rows.at[b], gsem)

    def stage_b(b):
        @plsc.parallel_loop(0, AC, unroll=8)
        def _(e):
            sp = plsc.load_gather(nrm_v.at[b], [jnp.full((16,), e, jnp.int32)])
            for k in range(8):
                rows[b, e, pl.ds(k * 16, 16)] = rows[b, e, pl.ds(k * 16, 16)] * sp
        return pltpu.async_copy(rows.at[b], acc.at[dst_v.at[b]], ssem, add=True)

    @pl.loop(0, tp // AC, step=2)
    def _(g0):
        cg0 = stage_a(g0, 0)
        cg1 = stage_a(g0 + 1, 1)
        cg0.wait()
        cs0 = stage_b(0)
        cg1.wait()
        cs1 = stage_b(1)
        cs0.wait()
        cs1.wait()

    plsc.subcore_barrier()
    pltpu.sync_copy(acc.at[pl.ds(s * (NACC // NS), NACC // NS)],
                    y_out.at[c, pl.ds(s * (NACC // NS), NACC // NS)])


def _agg_call(split_edges, tab, srcs, dsts, ews, dis1):
    mesh = plsc.VectorSubcoreMesh(core_axis_name="c", subcore_axis_name="s", num_cores=NC, num_subcores=NS)
    return pl.kernel(
        functools.partial(_agg_body, split_edges),
        out_type=jax.ShapeDtypeStruct((NC, NACC, 128), jnp.float32),
        mesh=mesh,
        scratch_types=[
            pltpu.VMEM((NACC,), jnp.float32),
            pltpu.VMEM((2, 128), jnp.int32),
            pltpu.VMEM((2, 128), jnp.int32),
            pltpu.VMEM((2, 128), jnp.int32),
            pltpu.VMEM((2, 128), jnp.float32),
            pltpu.VMEM((2, 128), jnp.float32),
            pltpu.VMEM((2, AC, 128), jnp.float32),
            pltpu.VMEM_SHARED((NACC, 128), jnp.float32),
            pltpu.SemaphoreType.DMA,
            pltpu.SemaphoreType.DMA,
        ],
        compiler_params=pltpu.CompilerParams(needs_layout_passes=False),
        name="sc_agg",
    )(tab, srcs, dsts, ews, dis1)


def _dis_body(deg_ref, dis_ref):
    d = jnp.sum(deg_ref[...], axis=0, keepdims=True) + 1.0
    dis_ref[...] = lax.rsqrt(d)


def _dis_call(degs):
    return pl.pallas_call(
        _dis_body,
        grid=(NACC // 512,),
        in_specs=[pl.BlockSpec((NC, 512), lambda i: (0, i))],
        out_specs=pl.BlockSpec((1, 512), lambda i: (0, i)),
        out_shape=jax.ShapeDtypeStruct((1, NACC), jnp.float32),
    )(degs)


_RB = 400


def _mid_body(y1_ref, x_ref, dis_ref, w1_ref, b1_ref, hs_ref):
    d = dis_ref[...]
    agg = y1_ref[0] + y1_ref[1] + (d * d) * x_ref[...]
    h = jnp.dot(agg, w1_ref[...], preferred_element_type=jnp.float32) + b1_ref[...]
    hs_ref[0] = jnp.maximum(h, 0.0)


def _mid_call(y1, x, dis2d, W1, b1r):
    return pl.pallas_call(
        _mid_body,
        grid=(N // _RB, 2),
        in_specs=[
            pl.BlockSpec((NC, _RB, 128), lambda i, c: (0, i, 0)),
            pl.BlockSpec((_RB, DIN), lambda i, c: (i, 0)),
            pl.BlockSpec((_RB, 1), lambda i, c: (i, 0)),
            pl.BlockSpec((DIN, 128), lambda i, c: (0, c)),
            pl.BlockSpec((1, 128), lambda i, c: (0, c)),
        ],
        out_specs=pl.BlockSpec((1, _RB, 128), lambda i, c: (c, i, 0)),
        out_shape=jax.ShapeDtypeStruct((2, N, 128), jnp.float32),
    )(y1, x, dis2d, W1, b1r)


def _fin_body(y2_ref, hs_ref, dis_ref, wmu_ref, bmu_ref, wls_ref, bls_ref,
              mu_ref, ls_ref):
    d = dis_ref[...]
    d2 = d * d
    aa = y2_ref[0] + d2 * hs_ref[0]
    ab = y2_ref[1] + d2 * hs_ref[1]
    mu_ref[...] = (
        jnp.dot(aa, wmu_ref[0:128], preferred_element_type=jnp.float32)
        + jnp.dot(ab, wmu_ref[128:256], preferred_element_type=jnp.float32)
        + bmu_ref[...]
    )
    ls_ref[...] = (
        jnp.dot(aa, wls_ref[0:128], preferred_element_type=jnp.float32)
        + jnp.dot(ab, wls_ref[128:256], preferred_element_type=jnp.float32)
        + bls_ref[...]
    )


def _fin_call(y2, hs, dis2d, Wmu, bmur, Wls, blsr):
    return pl.pallas_call(
        _fin_body,
        grid=(N // _RB,),
        in_specs=[
            pl.BlockSpec((NC, _RB, 128), lambda i: (0, i, 0)),
            pl.BlockSpec((NC, _RB, 128), lambda i: (0, i, 0)),
            pl.BlockSpec((_RB, 1), lambda i: (i, 0)),
            pl.BlockSpec((HID, OUT), lambda i: (0, 0)),
            pl.BlockSpec((1, OUT), lambda i: (0, 0)),
            pl.BlockSpec((HID, OUT), lambda i: (0, 0)),
            pl.BlockSpec((1, OUT), lambda i: (0, 0)),
        ],
        out_specs=[
            pl.BlockSpec((_RB, OUT), lambda i: (i, 0)),
            pl.BlockSpec((_RB, OUT), lambda i: (i, 0)),
        ],
        out_shape=[
            jax.ShapeDtypeStruct((N, OUT), jnp.float32),
            jax.ShapeDtypeStruct((N, OUT), jnp.float32),
        ],
    )(y2, hs, dis2d, Wmu, bmur, Wls, blsr)


@jax.jit
def kernel(x, edge_index, edge_attr, W1, b1, Wmu, bmu, Wls, bls):
    src = edge_index[0].astype(jnp.int32)
    dst = edge_index[1].astype(jnp.int32)
    pad = EP - E
    srcp = jnp.concatenate([src, jnp.zeros((pad,), jnp.int32)])
    dstp = jnp.concatenate([dst, jnp.full((pad,), N, jnp.int32)])
    ewp = jnp.concatenate([edge_attr, jnp.zeros((pad,), jnp.float32)])

    src1 = srcp.reshape(EP // 128, 128)
    dst2d = dstp.reshape(EP // 128, 128)

    degs = _deg_call(dst2d, ewp)
    dis = _dis_call(degs)
    dis1 = dis.reshape(NACC)
    dis2d = dis.reshape(NACC, 1)

    y1p = _agg_call(True, x, src1, dst2d, ewp, dis1)
    hs = _mid_call(y1p, x, dis2d[:N], W1, b1.reshape(1, HID))
    tab2 = hs.reshape(2 * N, 128)
    y2 = _agg_call(False, tab2, src1, dst2d, ewp, dis1)
    mu, ls = _fin_call(y2, hs, dis2d[:N], Wmu, bmu.reshape(1, OUT),
                       Wls, bls.reshape(1, OUT))
    return (mu, ls)


# speedup vs baseline: 15.1604x; 1.7936x over previous
import functools

import jax
import jax.numpy as jnp
from jax import lax
from jax.experimental import pallas as pl
from jax.experimental.pallas import tpu as pltpu
from jax.experimental.pallas import tpu_sc as plsc

N = 10000
E = 320000
EP = 327680
NACC = 10240
DIN = 128
HID = 256
OUT = 128
NC, NS = 2, 16
CHUNK = 256
G = CHUNK // 128
AC = 128


def _deg_body(dsts, ews, out, dst_v, ew_v, zbuf1, acc1):
    c = lax.axis_index("c")
    s = lax.axis_index("s")
    wid = c * NS + s
    tp = EP // (NC * NS)
    tpr = tp // 128
    nps = NACC // NS

    @pl.loop(0, nps // 16)
    def _(r):
        zbuf1[pl.ds(r * 16, 16)] = jnp.zeros((16,), jnp.float32)

    pltpu.sync_copy(zbuf1, acc1.at[pl.ds(s * nps, nps)])
    plsc.subcore_barrier()

    @pl.loop(0, tp // CHUNK)
    def _(g):
        pltpu.sync_copy(dsts.at[pl.ds(wid * tpr + g * G, G)], dst_v)
        pltpu.sync_copy(ews.at[pl.ds(wid * tp + g * CHUNK, CHUNK)], ew_v)
        for j in range(G):
            pltpu.sync_copy(ew_v.at[pl.ds(j * 128, 128)],
                            acc1.at[dst_v.at[j]], add=True)

    plsc.subcore_barrier()
    pltpu.sync_copy(acc1.at[pl.ds(s * nps, nps)],
                    out.at[c, pl.ds(s * nps, nps)])


def _deg_call(dst2d, ewp):
    mesh = plsc.VectorSubcoreMesh(core_axis_name="c", subcore_axis_name="s", num_cores=NC, num_subcores=NS)
    return pl.kernel(
        _deg_body,
        out_type=jax.ShapeDtypeStruct((NC, NACC), jnp.float32),
        mesh=mesh,
        scratch_types=[
            pltpu.VMEM((G, 128), jnp.int32),
            pltpu.VMEM((CHUNK,), jnp.float32),
            pltpu.VMEM((NACC // NS,), jnp.float32),
            pltpu.VMEM_SHARED((NACC,), jnp.float32),
        ],
        compiler_params=pltpu.CompilerParams(needs_layout_passes=False),
        name="sc_deg",
    )(dst2d, ewp)


def _agg_body(split_edges, tab, srcs, dsts, ews, dis_hbm, y_out,
              dis_v, src_v, dst_v, idx_v, ew_v, nrm_v, rows, acc, gsem, ssem):
    c = lax.axis_index("c")
    s = lax.axis_index("s")

    @pl.loop(0, AC)
    def _(r):
        for b in range(2):
            for k in range(8):
                rows[b, r, pl.ds(k * 16, 16)] = jnp.zeros((16,), jnp.float32)

    for j in range(NACC // NS // AC):
        pltpu.sync_copy(rows.at[j % 2],
                        acc.at[pl.ds(s * (NACC // NS) + j * AC, AC)])

    pltpu.sync_copy(dis_hbm, dis_v)
    plsc.subcore_barrier()

    if split_edges:
        tp = EP // (NC * NS)
        base = (c * NS + s) * tp
        srow = base // 128
        drow = srow
    else:
        tp = EP // NS
        base = s * tp
        srow = base // 128
        drow = srow
    off = 0 if split_edges else c * N

    def stage_a(g, b):
        pltpu.sync_copy(srcs.at[pl.ds(srow + g, 1)], src_v.at[pl.ds(b, 1)])
        pltpu.sync_copy(dsts.at[pl.ds(drow + g, 1)], dst_v.at[pl.ds(b, 1)])
        pltpu.sync_copy(ews.at[pl.ds(base + g * AC, AC)], ew_v.at[b])
        for i in range(AC // 16):
            s16 = src_v[b, pl.ds(i * 16, 16)]
            d16 = dst_v[b, pl.ds(i * 16, 16)]
            e16 = ew_v[b, pl.ds(i * 16, 16)]
            if not split_edges:
                idx_v[b, pl.ds(i * 16, 16)] = s16 + off
            nrm_v[b, pl.ds(i * 16, 16)] = (
                plsc.load_gather(dis_v, [s16]) * e16
                * plsc.load_gather(dis_v, [d16])
            )
        gidx = src_v if split_edges else idx_v
        return pltpu.async_copy(tab.at[gidx.at[b]], rows.at[b], gsem)

    def stage_b(b):
        @plsc.parallel_loop(0, AC, unroll=8)
        def _(e):
            sp = plsc.load_gather(nrm_v.at[b], [jnp.full((16,), e, jnp.int32)])
            for k in range(8):
                rows[b, e, pl.ds(k * 16, 16)] = rows[b, e, pl.ds(k * 16, 16)] * sp
        return pltpu.async_copy(rows.at[b], acc.at[dst_v.at[b]], ssem, add=True)

    @pl.loop(0, tp // AC, step=2)
    def _(g0):
        cg0 = stage_a(g0, 0)
        cg1 = stage_a(g0 + 1, 1)
        cg0.wait()
        cs0 = stage_b(0)
        cg1.wait()
        cs1 = stage_b(1)
        cs0.wait()
        cs1.wait()

    plsc.subcore_barrier()
    pltpu.sync_copy(acc.at[pl.ds(s * (NACC // NS), NACC // NS)],
                    y_out.at[c, pl.ds(s * (NACC // NS), NACC // NS)])


def _agg_call(split_edges, tab, srcs, dsts, ews, dis1):
    mesh = plsc.VectorSubcoreMesh(core_axis_name="c", subcore_axis_name="s", num_cores=NC, num_subcores=NS)
    return pl.kernel(
        functools.partial(_agg_body, split_edges),
        out_type=jax.ShapeDtypeStruct((NC, NACC, 128), jnp.float32),
        mesh=mesh,
        scratch_types=[
            pltpu.VMEM((NACC,), jnp.float32),
            pltpu.VMEM((2, 128), jnp.int32),
            pltpu.VMEM((2, 128), jnp.int32),
            pltpu.VMEM((2, 128), jnp.int32),
            pltpu.VMEM((2, 128), jnp.float32),
            pltpu.VMEM((2, 128), jnp.float32),
            pltpu.VMEM((2, AC, 128), jnp.float32),
            pltpu.VMEM_SHARED((NACC, 128), jnp.float32),
            pltpu.SemaphoreType.DMA,
            pltpu.SemaphoreType.DMA,
        ],
        compiler_params=pltpu.CompilerParams(needs_layout_passes=False),
        name="sc_agg",
    )(tab, srcs, dsts, ews, dis1)


def _dis_body(deg_ref, dis_ref):
    d = jnp.sum(deg_ref[...], axis=0, keepdims=True) + 1.0
    dis_ref[...] = lax.rsqrt(d)


def _dis_call(degs):
    return pl.pallas_call(
        _dis_body,
        grid=(NACC // 512,),
        in_specs=[pl.BlockSpec((NC, 512), lambda i: (0, i))],
        out_specs=pl.BlockSpec((1, 512), lambda i: (0, i)),
        out_shape=jax.ShapeDtypeStruct((1, NACC), jnp.float32),
    )(degs)


_RB = 400


def _mid_body(y1_ref, x_ref, dis_ref, w1_ref, b1_ref, hs_ref):
    d = dis_ref[...]
    agg = y1_ref[0] + y1_ref[1] + (d * d) * x_ref[...]
    h = jnp.dot(agg, w1_ref[...], preferred_element_type=jnp.float32) + b1_ref[...]
    hs_ref[0] = jnp.maximum(h, 0.0)


def _mid_call(y1, x, dis2d, W1, b1r):
    return pl.pallas_call(
        _mid_body,
        grid=(N // _RB, 2),
        in_specs=[
            pl.BlockSpec((NC, _RB, 128), lambda i, c: (0, i, 0)),
            pl.BlockSpec((_RB, DIN), lambda i, c: (i, 0)),
            pl.BlockSpec((_RB, 1), lambda i, c: (i, 0)),
            pl.BlockSpec((DIN, 128), lambda i, c: (0, c)),
            pl.BlockSpec((1, 128), lambda i, c: (0, c)),
        ],
        out_specs=pl.BlockSpec((1, _RB, 128), lambda i, c: (c, i, 0)),
        out_shape=jax.ShapeDtypeStruct((2, N, 128), jnp.float32),
    )(y1, x, dis2d, W1, b1r)


def _fin_body(y2_ref, hs_ref, dis_ref, wmu_ref, bmu_ref, wls_ref, bls_ref,
              mu_ref, ls_ref):
    d = dis_ref[...]
    d2 = d * d
    aa = y2_ref[0] + d2 * hs_ref[0]
    ab = y2_ref[1] + d2 * hs_ref[1]
    mu_ref[...] = (
        jnp.dot(aa, wmu_ref[0:128], preferred_element_type=jnp.float32)
        + jnp.dot(ab, wmu_ref[128:256], preferred_element_type=jnp.float32)
        + bmu_ref[...]
    )
    ls_ref[...] = (
        jnp.dot(aa, wls_ref[0:128], preferred_element_type=jnp.float32)
        + jnp.dot(ab, wls_ref[128:256], preferred_element_type=jnp.float32)
        + bls_ref[...]
    )


def _fin_call(y2, hs, dis2d, Wmu, bmur, Wls, blsr):
    return pl.pallas_call(
        _fin_body,
        grid=(N // _RB,),
        in_specs=[
            pl.BlockSpec((NC, _RB, 128), lambda i: (0, i, 0)),
            pl.BlockSpec((NC, _RB, 128), lambda i: (0, i, 0)),
            pl.BlockSpec((_RB, 1), lambda i: (i, 0)),
            pl.BlockSpec((HID, OUT), lambda i: (0, 0)),
            pl.BlockSpec((1, OUT), lambda i: (0, 0)),
            pl.BlockSpec((HID, OUT), lambda i: (0, 0)),
            pl.BlockSpec((1, OUT), lambda i: (0, 0)),
        ],
        out_specs=[
            pl.BlockSpec((_RB, OUT), lambda i: (i, 0)),
            pl.BlockSpec((_RB, OUT), lambda i: (i, 0)),
        ],
        out_shape=[
            jax.ShapeDtypeStruct((N, OUT), jnp.float32),
            jax.ShapeDtypeStruct((N, OUT), jnp.float32),
        ],
    )(y2, hs, dis2d, Wmu, bmur, Wls, blsr)


@jax.jit
def kernel(x, edge_index, edge_attr, W1, b1, Wmu, bmu, Wls, bls):
    src = edge_index[0].astype(jnp.int32)
    dst = edge_index[1].astype(jnp.int32)
    pad = EP - E
    ar = jnp.arange(pad, dtype=jnp.int32)
    srcp = jnp.concatenate([src, ar % N])
    dstp = jnp.concatenate([dst, N + (ar % (NACC - N))])
    ewp = jnp.concatenate([edge_attr, jnp.zeros((pad,), jnp.float32)])

    src1 = srcp.reshape(EP // 128, 128)
    dst2d = dstp.reshape(EP // 128, 128)

    degs = _deg_call(dst2d, ewp)
    dis = _dis_call(degs)
    dis1 = dis.reshape(NACC)
    dis2d = dis.reshape(NACC, 1)

    y1p = _agg_call(True, x, src1, dst2d, ewp, dis1)
    hs = _mid_call(y1p, x, dis2d[:N], W1, b1.reshape(1, HID))
    tab2 = hs.reshape(2 * N, 128)
    y2 = _agg_call(False, tab2, src1, dst2d, ewp, dis1)
    mu, ls = _fin_call(y2, hs, dis2d[:N], Wmu, bmu.reshape(1, OUT),
                       Wls, bls.reshape(1, OUT))
    return (mu, ls)


# speedup vs baseline: 16.4957x; 1.0881x over previous
import functools

import jax
import jax.numpy as jnp
from jax import lax
from jax.experimental import pallas as pl
from jax.experimental.pallas import tpu as pltpu
from jax.experimental.pallas import tpu_sc as plsc

N = 10000
E = 320000
EP = 327680
NACC = 10240
DIN = 128
HID = 256
OUT = 128
NC, NS = 2, 16
CHUNK = 256
G = CHUNK // 128
AC = 128


def _deg_body(dsts, ews, out, dst_v, ew_v, zbuf1, acc1):
    c = lax.axis_index("c")
    s = lax.axis_index("s")
    wid = c * NS + s
    tp = EP // (NC * NS)
    tpr = tp // 128
    nps = NACC // NS

    @pl.loop(0, nps // 16)
    def _(r):
        zbuf1[pl.ds(r * 16, 16)] = jnp.zeros((16,), jnp.float32)

    pltpu.sync_copy(zbuf1, acc1.at[pl.ds(s * nps, nps)])
    plsc.subcore_barrier()

    @pl.loop(0, tp // CHUNK)
    def _(g):
        pltpu.sync_copy(dsts.at[pl.ds(wid * tpr + g * G, G)], dst_v)
        pltpu.sync_copy(ews.at[pl.ds(wid * tp + g * CHUNK, CHUNK)], ew_v)
        for j in range(G):
            pltpu.sync_copy(ew_v.at[pl.ds(j * 128, 128)],
                            acc1.at[dst_v.at[j]], add=True)

    plsc.subcore_barrier()
    pltpu.sync_copy(acc1.at[pl.ds(s * nps, nps)],
                    out.at[c, pl.ds(s * nps, nps)])


def _deg_call(dst2d, ewp):
    mesh = plsc.VectorSubcoreMesh(core_axis_name="c", subcore_axis_name="s", num_cores=NC, num_subcores=NS)
    return pl.kernel(
        _deg_body,
        out_type=jax.ShapeDtypeStruct((NC, NACC), jnp.float32),
        mesh=mesh,
        scratch_types=[
            pltpu.VMEM((G, 128), jnp.int32),
            pltpu.VMEM((CHUNK,), jnp.float32),
            pltpu.VMEM((NACC // NS,), jnp.float32),
            pltpu.VMEM_SHARED((NACC,), jnp.float32),
        ],
        compiler_params=pltpu.CompilerParams(needs_layout_passes=False),
        name="sc_deg",
    )(dst2d, ewp)


def _agg_body(split_edges, tab, srcs, dsts, ews, dis_hbm, y_out,
              dis_v, src_v, dst_v, idx_v, ew_v, nrm_v, rows, acc,
              gsem, ssem0, ssem1):
    c = lax.axis_index("c")
    s = lax.axis_index("s")
    ssem = (ssem0, ssem1)

    @pl.loop(0, AC)
    def _(r):
        for b in range(2):
            for k in range(8):
                rows[b, r, pl.ds(k * 16, 16)] = jnp.zeros((16,), jnp.float32)

    for j in range(NACC // NS // AC):
        pltpu.sync_copy(rows.at[j % 2],
                        acc.at[pl.ds(s * (NACC // NS) + j * AC, AC)])

    pltpu.sync_copy(dis_hbm, dis_v)
    plsc.subcore_barrier()

    if split_edges:
        tp = EP // (NC * NS)
        base = (c * NS + s) * tp
        srow = base // 128
        drow = srow
    else:
        tp = EP // NS
        base = s * tp
        srow = base // 128
        drow = srow
    off = 0 if split_edges else c * N

    def stage_a(g, b):
        pltpu.sync_copy(srcs.at[pl.ds(srow + g, 1)], src_v.at[pl.ds(b, 1)])
        pltpu.sync_copy(dsts.at[pl.ds(drow + g, 1)], dst_v.at[pl.ds(b, 1)])
        pltpu.sync_copy(ews.at[pl.ds(base + g * AC, AC)], ew_v.at[b])
        for i in range(AC // 16):
            s16 = src_v[b, pl.ds(i * 16, 16)]
            d16 = dst_v[b, pl.ds(i * 16, 16)]
            e16 = ew_v[b, pl.ds(i * 16, 16)]
            if not split_edges:
                idx_v[b, pl.ds(i * 16, 16)] = s16 + off
            nrm_v[b, pl.ds(i * 16, 16)] = (
                plsc.load_gather(dis_v, [s16]) * e16
                * plsc.load_gather(dis_v, [d16])
            )
        gidx = src_v if split_edges else idx_v
        return pltpu.async_copy(tab.at[gidx.at[b]], rows.at[b], gsem)

    def stage_b(b):
        @plsc.parallel_loop(0, AC, unroll=8)
        def _(e):
            sp = plsc.load_gather(nrm_v.at[b], [jnp.full((16,), e, jnp.int32)])
            for k in range(8):
                rows[b, e, pl.ds(k * 16, 16)] = rows[b, e, pl.ds(k * 16, 16)] * sp
        return pltpu.async_copy(rows.at[b], acc.at[dst_v.at[b]],
                                ssem[b], add=True)

    def drain_scatter(b):
        pltpu.make_async_copy(rows.at[b], acc.at[dst_v.at[b]], ssem[b]).wait()

    cg0 = stage_a(0, 0)
    cg1 = stage_a(1, 1)
    cg0.wait()
    stage_b(0)
    cg1.wait()
    stage_b(1)

    @pl.loop(2, tp // AC, step=2)
    def _(g0):
        drain_scatter(0)
        cg0 = stage_a(g0, 0)
        drain_scatter(1)
        cg1 = stage_a(g0 + 1, 1)
        cg0.wait()
        stage_b(0)
        cg1.wait()
        stage_b(1)

    drain_scatter(0)
    drain_scatter(1)

    plsc.subcore_barrier()
    pltpu.sync_copy(acc.at[pl.ds(s * (NACC // NS), NACC // NS)],
                    y_out.at[c, pl.ds(s * (NACC // NS), NACC // NS)])


def _agg_call(split_edges, tab, srcs, dsts, ews, dis1):
    mesh = plsc.VectorSubcoreMesh(core_axis_name="c", subcore_axis_name="s", num_cores=NC, num_subcores=NS)
    return pl.kernel(
        functools.partial(_agg_body, split_edges),
        out_type=jax.ShapeDtypeStruct((NC, NACC, 128), jnp.float32),
        mesh=mesh,
        scratch_types=[
            pltpu.VMEM((NACC,), jnp.float32),
            pltpu.VMEM((2, 128), jnp.int32),
            pltpu.VMEM((2, 128), jnp.int32),
            pltpu.VMEM((2, 128), jnp.int32),
            pltpu.VMEM((2, 128), jnp.float32),
            pltpu.VMEM((2, 128), jnp.float32),
            pltpu.VMEM((2, AC, 128), jnp.float32),
            pltpu.VMEM_SHARED((NACC, 128), jnp.float32),
            pltpu.SemaphoreType.DMA,
            pltpu.SemaphoreType.DMA,
            pltpu.SemaphoreType.DMA,
        ],
        compiler_params=pltpu.CompilerParams(needs_layout_passes=False),
        name="sc_agg",
    )(tab, srcs, dsts, ews, dis1)


def _dis_body(deg_ref, dis_ref):
    d = jnp.sum(deg_ref[...], axis=0, keepdims=True) + 1.0
    dis_ref[...] = lax.rsqrt(d)


def _dis_call(degs):
    return pl.pallas_call(
        _dis_body,
        grid=(NACC // 512,),
        in_specs=[pl.BlockSpec((NC, 512), lambda i: (0, i))],
        out_specs=pl.BlockSpec((1, 512), lambda i: (0, i)),
        out_shape=jax.ShapeDtypeStruct((1, NACC), jnp.float32),
    )(degs)


_RB = 400


def _mid_body(y1_ref, x_ref, dis_ref, w1_ref, b1_ref, hs_ref):
    d = dis_ref[...]
    agg = y1_ref[0] + y1_ref[1] + (d * d) * x_ref[...]
    h = jnp.dot(agg, w1_ref[...], preferred_element_type=jnp.float32) + b1_ref[...]
    hs_ref[0] = jnp.maximum(h, 0.0)


def _mid_call(y1, x, dis2d, W1, b1r):
    return pl.pallas_call(
        _mid_body,
        grid=(N // _RB, 2),
        in_specs=[
            pl.BlockSpec((NC, _RB, 128), lambda i, c: (0, i, 0)),
            pl.BlockSpec((_RB, DIN), lambda i, c: (i, 0)),
            pl.BlockSpec((_RB, 1), lambda i, c: (i, 0)),
            pl.BlockSpec((DIN, 128), lambda i, c: (0, c)),
            pl.BlockSpec((1, 128), lambda i, c: (0, c)),
        ],
        out_specs=pl.BlockSpec((1, _RB, 128), lambda i, c: (c, i, 0)),
        out_shape=jax.ShapeDtypeStruct((2, N, 128), jnp.float32),
    )(y1, x, dis2d, W1, b1r)


def _fin_body(y2_ref, hs_ref, dis_ref, wmu_ref, bmu_ref, wls_ref, bls_ref,
              mu_ref, ls_ref):
    d = dis_ref[...]
    d2 = d * d
    aa = y2_ref[0] + d2 * hs_ref[0]
    ab = y2_ref[1] + d2 * hs_ref[1]
    mu_ref[...] = (
        jnp.dot(aa, wmu_ref[0:128], preferred_element_type=jnp.float32)
        + jnp.dot(ab, wmu_ref[128:256], preferred_element_type=jnp.float32)
        + bmu_ref[...]
    )
    ls_ref[...] = (
        jnp.dot(aa, wls_ref[0:128], preferred_element_type=jnp.float32)
        + jnp.dot(ab, wls_ref[128:256], preferred_element_type=jnp.float32)
        + bls_ref[...]
    )


def _fin_call(y2, hs, dis2d, Wmu, bmur, Wls, blsr):
    return pl.pallas_call(
        _fin_body,
        grid=(N // _RB,),
        in_specs=[
            pl.BlockSpec((NC, _RB, 128), lambda i: (0, i, 0)),
            pl.BlockSpec((NC, _RB, 128), lambda i: (0, i, 0)),
            pl.BlockSpec((_RB, 1), lambda i: (i, 0)),
            pl.BlockSpec((HID, OUT), lambda i: (0, 0)),
            pl.BlockSpec((1, OUT), lambda i: (0, 0)),
            pl.BlockSpec((HID, OUT), lambda i: (0, 0)),
            pl.BlockSpec((1, OUT), lambda i: (0, 0)),
        ],
        out_specs=[
            pl.BlockSpec((_RB, OUT), lambda i: (i, 0)),
            pl.BlockSpec((_RB, OUT), lambda i: (i, 0)),
        ],
        out_shape=[
            jax.ShapeDtypeStruct((N, OUT), jnp.float32),
            jax.ShapeDtypeStruct((N, OUT), jnp.float32),
        ],
    )(y2, hs, dis2d, Wmu, bmur, Wls, blsr)


@jax.jit
def kernel(x, edge_index, edge_attr, W1, b1, Wmu, bmu, Wls, bls):
    src = edge_index[0].astype(jnp.int32)
    dst = edge_index[1].astype(jnp.int32)
    pad = EP - E
    ar = jnp.arange(pad, dtype=jnp.int32)
    srcp = jnp.concatenate([src, ar % N])
    dstp = jnp.concatenate([dst, N + (ar % (NACC - N))])
    ewp = jnp.concatenate([edge_attr, jnp.zeros((pad,), jnp.float32)])

    src1 = srcp.reshape(EP // 128, 128)
    dst2d = dstp.reshape(EP // 128, 128)

    degs = _deg_call(dst2d, ewp)
    dis = _dis_call(degs)
    dis1 = dis.reshape(NACC)
    dis2d = dis.reshape(NACC, 1)

    y1p = _agg_call(True, x, src1, dst2d, ewp, dis1)
    hs = _mid_call(y1p, x, dis2d[:N], W1, b1.reshape(1, HID))
    tab2 = hs.reshape(2 * N, 128)
    y2 = _agg_call(False, tab2, src1, dst2d, ewp, dis1)
    mu, ls = _fin_call(y2, hs, dis2d[:N], Wmu, bmu.reshape(1, OUT),
                       Wls, bls.reshape(1, OUT))
    return (mu, ls)


# speedup vs baseline: 16.9486x; 1.0275x over previous
import functools

import jax
import jax.numpy as jnp
from jax import lax
from jax.experimental import pallas as pl
from jax.experimental.pallas import tpu as pltpu
from jax.experimental.pallas import tpu_sc as plsc

N = 10000
E = 320000
EP = 327680
NACC = 10240
DIN = 128
HID = 256
OUT = 128
NC, NS = 2, 16
CHUNK = 256
G = CHUNK // 128
AC = 128


def _deg_body(dsts, ews, out, dst_v, ew_v, zbuf1, acc1, lsem, ssem):
    c = lax.axis_index("c")
    s = lax.axis_index("s")
    wid = c * NS + s
    tp = EP // (NC * NS)
    tpr = tp // 128
    nps = NACC // NS

    @pl.loop(0, nps // 16)
    def _(r):
        zbuf1[pl.ds(r * 16, 16)] = jnp.zeros((16,), jnp.float32)

    pltpu.sync_copy(zbuf1, acc1.at[pl.ds(s * nps, nps)])
    plsc.subcore_barrier()

    @pl.loop(0, tp // CHUNK)
    def _(g):
        cl0 = pltpu.async_copy(dsts.at[pl.ds(wid * tpr + g * G, G)],
                               dst_v, lsem)
        cl1 = pltpu.async_copy(ews.at[pl.ds(wid * tp + g * CHUNK, CHUNK)],
                               ew_v, lsem)
        cl0.wait()
        cl1.wait()
        cps = [
            pltpu.async_copy(ew_v.at[pl.ds(j * 128, 128)],
                             acc1.at[dst_v.at[j]], ssem, add=True)
            for j in range(G)
        ]
        for cp in cps:
            cp.wait()

    plsc.subcore_barrier()
    pltpu.sync_copy(acc1.at[pl.ds(s * nps, nps)],
                    out.at[c, pl.ds(s * nps, nps)])


def _deg_call(dst2d, ewp):
    mesh = plsc.VectorSubcoreMesh(core_axis_name="c", subcore_axis_name="s", num_cores=NC, num_subcores=NS)
    return pl.kernel(
        _deg_body,
        out_type=jax.ShapeDtypeStruct((NC, NACC), jnp.float32),
        mesh=mesh,
        scratch_types=[
            pltpu.VMEM((G, 128), jnp.int32),
            pltpu.VMEM((CHUNK,), jnp.float32),
            pltpu.VMEM((NACC // NS,), jnp.float32),
            pltpu.VMEM_SHARED((NACC,), jnp.float32),
            pltpu.SemaphoreType.DMA,
            pltpu.SemaphoreType.DMA,
        ],
        compiler_params=pltpu.CompilerParams(needs_layout_passes=False),
        name="sc_deg",
    )(dst2d, ewp)


def _agg_body(split_edges, tab, srcs, dsts, ews, dis_hbm, y_out,
              dis_v, src_v, dst_v, idx_v, ew_v, nrm_v, rows, acc,
              gsem, ssem0, ssem1):
    c = lax.axis_index("c")
    s = lax.axis_index("s")
    ssem = (ssem0, ssem1)

    @pl.loop(0, AC)
    def _(r):
        for b in range(2):
            for k in range(8):
                rows[b, r, pl.ds(k * 16, 16)] = jnp.zeros((16,), jnp.float32)

    for j in range(NACC // NS // AC):
        pltpu.sync_copy(rows.at[j % 2],
                        acc.at[pl.ds(s * (NACC // NS) + j * AC, AC)])

    pltpu.sync_copy(dis_hbm, dis_v)
    plsc.subcore_barrier()

    if split_edges:
        tp = EP // (NC * NS)
        base = (c * NS + s) * tp
        srow = base // 128
        drow = srow
    else:
        tp = EP // NS
        base = s * tp
        srow = base // 128
        drow = srow
    off = 0 if split_edges else c * N

    def stage_a(g, b):
        pltpu.sync_copy(srcs.at[pl.ds(srow + g, 1)], src_v.at[pl.ds(b, 1)])
        pltpu.sync_copy(dsts.at[pl.ds(drow + g, 1)], dst_v.at[pl.ds(b, 1)])
        pltpu.sync_copy(ews.at[pl.ds(base + g * AC, AC)], ew_v.at[b])
        for i in range(AC // 16):
            s16 = src_v[b, pl.ds(i * 16, 16)]
            d16 = dst_v[b, pl.ds(i * 16, 16)]
            e16 = ew_v[b, pl.ds(i * 16, 16)]
            if not split_edges:
                idx_v[b, pl.ds(i * 16, 16)] = s16 + off
            nrm_v[b, pl.ds(i * 16, 16)] = (
                plsc.load_gather(dis_v, [s16]) * e16
                * plsc.load_gather(dis_v, [d16])
            )
        gidx = src_v if split_edges else idx_v
        return pltpu.async_copy(tab.at[gidx.at[b]], rows.at[b], gsem)

    def stage_b(b):
        @plsc.parallel_loop(0, AC, unroll=8)
        def _(e):
            sp = plsc.load_gather(nrm_v.at[b], [jnp.full((16,), e, jnp.int32)])
            for k in range(8):
                rows[b, e, pl.ds(k * 16, 16)] = rows[b, e, pl.ds(k * 16, 16)] * sp
        return pltpu.async_copy(rows.at[b], acc.at[dst_v.at[b]],
                                ssem[b], add=True)

    def drain_scatter(b):
        pltpu.make_async_copy(rows.at[b], acc.at[dst_v.at[b]], ssem[b]).wait()

    cg0 = stage_a(0, 0)
    cg1 = stage_a(1, 1)
    cg0.wait()
    stage_b(0)
    cg1.wait()
    stage_b(1)

    @pl.loop(2, tp // AC, step=2)
    def _(g0):
        drain_scatter(0)
        cg0 = stage_a(g0, 0)
        drain_scatter(1)
        cg1 = stage_a(g0 + 1, 1)
        cg0.wait()
        stage_b(0)
        cg1.wait()
        stage_b(1)

    drain_scatter(0)
    drain_scatter(1)

    plsc.subcore_barrier()
    pltpu.sync_copy(acc.at[pl.ds(s * (NACC // NS), NACC // NS)],
                    y_out.at[c, pl.ds(s * (NACC // NS), NACC // NS)])


def _agg_call(split_edges, tab, srcs, dsts, ews, dis1):
    mesh = plsc.VectorSubcoreMesh(core_axis_name="c", subcore_axis_name="s", num_cores=NC, num_subcores=NS)
    return pl.kernel(
        functools.partial(_agg_body, split_edges),
        out_type=jax.ShapeDtypeStruct((NC, NACC, 128), jnp.float32),
        mesh=mesh,
        scratch_types=[
            pltpu.VMEM((NACC,), jnp.float32),
            pltpu.VMEM((2, 128), jnp.int32),
            pltpu.VMEM((2, 128), jnp.int32),
            pltpu.VMEM((2, 128), jnp.int32),
            pltpu.VMEM((2, 128), jnp.float32),
            pltpu.VMEM((2, 128), jnp.float32),
            pltpu.VMEM((2, AC, 128), jnp.float32),
            pltpu.VMEM_SHARED((NACC, 128), jnp.float32),
            pltpu.SemaphoreType.DMA,
            pltpu.SemaphoreType.DMA,
            pltpu.SemaphoreType.DMA,
        ],
        compiler_params=pltpu.CompilerParams(needs_layout_passes=False),
        name="sc_agg",
    )(tab, srcs, dsts, ews, dis1)


def _dis_body(deg_ref, dis_ref):
    d = jnp.sum(deg_ref[...], axis=0, keepdims=True) + 1.0
    dis_ref[...] = lax.rsqrt(d)


def _dis_call(degs):
    return pl.pallas_call(
        _dis_body,
        grid=(NACC // 512,),
        in_specs=[pl.BlockSpec((NC, 512), lambda i: (0, i))],
        out_specs=pl.BlockSpec((1, 512), lambda i: (0, i)),
        out_shape=jax.ShapeDtypeStruct((1, NACC), jnp.float32),
    )(degs)


_RB = 400


def _mid_body(y1_ref, x_ref, dis_ref, w1_ref, b1_ref, hs_ref):
    d = dis_ref[...]
    agg = y1_ref[0] + y1_ref[1] + (d * d) * x_ref[...]
    h = jnp.dot(agg, w1_ref[...], preferred_element_type=jnp.float32) + b1_ref[...]
    hs_ref[0] = jnp.maximum(h, 0.0)


def _mid_call(y1, x, dis2d, W1, b1r):
    return pl.pallas_call(
        _mid_body,
        grid=(N // _RB, 2),
        in_specs=[
            pl.BlockSpec((NC, _RB, 128), lambda i, c: (0, i, 0)),
            pl.BlockSpec((_RB, DIN), lambda i, c: (i, 0)),
            pl.BlockSpec((_RB, 1), lambda i, c: (i, 0)),
            pl.BlockSpec((DIN, 128), lambda i, c: (0, c)),
            pl.BlockSpec((1, 128), lambda i, c: (0, c)),
        ],
        out_specs=pl.BlockSpec((1, _RB, 128), lambda i, c: (c, i, 0)),
        out_shape=jax.ShapeDtypeStruct((2, N, 128), jnp.float32),
    )(y1, x, dis2d, W1, b1r)


def _fin_body(y2_ref, hs_ref, dis_ref, wmu_ref, bmu_ref, wls_ref, bls_ref,
              mu_ref, ls_ref):
    d = dis_ref[...]
    d2 = d * d
    aa = y2_ref[0] + d2 * hs_ref[0]
    ab = y2_ref[1] + d2 * hs_ref[1]
    mu_ref[...] = (
        jnp.dot(aa, wmu_ref[0:128], preferred_element_type=jnp.float32)
        + jnp.dot(ab, wmu_ref[128:256], preferred_element_type=jnp.float32)
        + bmu_ref[...]
    )
    ls_ref[...] = (
        jnp.dot(aa, wls_ref[0:128], preferred_element_type=jnp.float32)
        + jnp.dot(ab, wls_ref[128:256], preferred_element_type=jnp.float32)
        + bls_ref[...]
    )


def _fin_call(y2, hs, dis2d, Wmu, bmur, Wls, blsr):
    return pl.pallas_call(
        _fin_body,
        grid=(N // _RB,),
        in_specs=[
            pl.BlockSpec((NC, _RB, 128), lambda i: (0, i, 0)),
            pl.BlockSpec((NC, _RB, 128), lambda i: (0, i, 0)),
            pl.BlockSpec((_RB, 1), lambda i: (i, 0)),
            pl.BlockSpec((HID, OUT), lambda i: (0, 0)),
            pl.BlockSpec((1, OUT), lambda i: (0, 0)),
            pl.BlockSpec((HID, OUT), lambda i: (0, 0)),
            pl.BlockSpec((1, OUT), lambda i: (0, 0)),
        ],
        out_specs=[
            pl.BlockSpec((_RB, OUT), lambda i: (i, 0)),
            pl.BlockSpec((_RB, OUT), lambda i: (i, 0)),
        ],
        out_shape=[
            jax.ShapeDtypeStruct((N, OUT), jnp.float32),
            jax.ShapeDtypeStruct((N, OUT), jnp.float32),
        ],
    )(y2, hs, dis2d, Wmu, bmur, Wls, blsr)


@jax.jit
def kernel(x, edge_index, edge_attr, W1, b1, Wmu, bmu, Wls, bls):
    src = edge_index[0].astype(jnp.int32)
    dst = edge_index[1].astype(jnp.int32)
    pad = EP - E
    ar = jnp.arange(pad, dtype=jnp.int32)
    srcp = jnp.concatenate([src, ar % N])
    dstp = jnp.concatenate([dst, N + (ar % (NACC - N))])
    ewp = jnp.concatenate([edge_attr, jnp.zeros((pad,), jnp.float32)])

    src1 = srcp.reshape(EP // 128, 128)
    dst2d = dstp.reshape(EP // 128, 128)

    degs = _deg_call(dst2d, ewp)
    dis = _dis_call(degs)
    dis1 = dis.reshape(NACC)
    dis2d = dis.reshape(NACC, 1)

    y1p = _agg_call(True, x, src1, dst2d, ewp, dis1)
    hs = _mid_call(y1p, x, dis2d[:N], W1, b1.reshape(1, HID))
    tab2 = hs.reshape(2 * N, 128)
    y2 = _agg_call(False, tab2, src1, dst2d, ewp, dis1)
    mu, ls = _fin_call(y2, hs, dis2d[:N], Wmu, bmu.reshape(1, OUT),
                       Wls, bls.reshape(1, OUT))
    return (mu, ls)
